# baseline (speedup 1.0000x reference)
import functools

import jax
import jax.numpy as jnp
from jax import lax
from jax.experimental import pallas as pl
from jax.experimental.pallas import tpu as pltpu
from jax.experimental.pallas import tpu_sc as plsc

NC = 2
NS = 16
NW = NC * NS
CH = 128
NG = 64


def _make_agg(n_pad, h, ncw, hbm_gather_frac4=0):
    mesh = plsc.VectorSubcoreMesh(
        core_axis_name="c", subcore_axis_name="s", num_cores=NC, num_subcores=NS
    )
    rpt = n_pad // NS

    @functools.partial(
        pl.kernel,
        out_type=jax.ShapeDtypeStruct((NC, n_pad, h), jnp.float32),
        mesh=mesh,
        scratch_types=[
            pltpu.VMEM((ncw, CH), jnp.int32),
            pltpu.VMEM((ncw, CH), jnp.int32),
            pltpu.VMEM((2, CH, h), jnp.float32),
            pltpu.VMEM_SHARED((n_pad, h), jnp.float32),
            pltpu.VMEM_SHARED((n_pad, h), jnp.float32),
        ] + [pltpu.SemaphoreType.DMA] * 4,
        compiler_params=pltpu.CompilerParams(use_tc_tiling_on_sc=False),
    )
    def agg(tab, srcp, dstp, zros, out, src_v, dst_v, rows_v, tab_sh, acc,
            g0, g1, s0, s1):
        c = lax.axis_index("c")
        s = lax.axis_index("s")
        w = s * NC + c
        gsem = (g0, g1)
        ssem = (s0, s1)
        pltpu.sync_copy(srcp.at[w], src_v)
        pltpu.sync_copy(dstp.at[w], dst_v)
        pltpu.sync_copy(tab.at[pl.ds(s * rpt, rpt)], tab_sh.at[pl.ds(s * rpt, rpt)])
        pltpu.sync_copy(zros.at[pl.ds(s * rpt, rpt)], acc.at[pl.ds(s * rpt, rpt)])
        plsc.subcore_barrier()

        def src_tab(t):
            return tab if t % 4 < hbm_gather_frac4 else tab_sh

        def gather(i, b, t):
            pltpu.async_copy(src_tab(t).at[src_v.at[i]], rows_v.at[b], gsem[b])

        def gather_wait(i, b, t):
            pltpu.make_async_copy(
                src_tab(t).at[src_v.at[i]], rows_v.at[b], gsem[b]
            ).wait()

        def scatter(i, b):
            pltpu.async_copy(rows_v.at[b], acc.at[dst_v.at[i]], ssem[b], add=True)

        def scatter_wait(i, b):
            pltpu.make_async_copy(rows_v.at[b], acc.at[dst_v.at[i]], ssem[b]).wait()

        for b in (0, 1):
            gather(b, b, b)

        def step(j, carry):
            i0 = 4 * j
            for half in (0, 2):
                for t in (half, half + 1):
                    b = t % 2
                    gather_wait(i0 + t, b, t)
                    scatter(i0 + t, b)
                for t in (half, half + 1):
                    b = t % 2
                    scatter_wait(i0 + t, b)

                    @pl.when(i0 + t + 2 < ncw)
                    def _():
                        gather(i0 + t + 2, b, t + 2)

            return carry

        lax.fori_loop(0, ncw // 4, step, 0)
        plsc.subcore_barrier()
        pltpu.sync_copy(
            acc.at[pl.ds(s * rpt, rpt)], out.at[c, pl.ds(s * rpt, rpt)]
        )

    return agg


def _make_deg(n_pad, ncw):
    mesh = plsc.VectorSubcoreMesh(
        core_axis_name="c", subcore_axis_name="s", num_cores=NC, num_subcores=NS
    )
    epw = ncw * CH

    @functools.partial(
        pl.kernel,
        out_type=jax.ShapeDtypeStruct((NW, n_pad), jnp.float32),
        mesh=mesh,
        scratch_types=[
            pltpu.VMEM((epw,), jnp.int32),
            pltpu.VMEM((n_pad,), jnp.float32),
        ],
        compiler_params=pltpu.CompilerParams(
            use_tc_tiling_on_sc=False, needs_layout_passes=False
        ),
    )
    def deg(dstp, zros, out, idx_v, hist_v):
        c = lax.axis_index("c")
        s = lax.axis_index("s")
        w = s * NC + c
        pltpu.sync_copy(dstp.at[w], idx_v)
        pltpu.sync_copy(zros, hist_v)
        ones = jnp.ones((16,), jnp.float32)

        def step(k, carry):
            for u in range(8):
                idx16 = idx_v[pl.ds((8 * k + u) * 16, 16)]
                plsc.addupdate_scatter(hist_v, [idx16], ones)
            return carry

        lax.fori_loop(0, epw // 128, step, 0)
        pltpu.sync_copy(hist_v, out.at[w])

    return deg


def _tc_first(x_ref, w_ref, degp_ref, y_ref, dis_ref, *, n, n_pad, h):
    deg = 1.0 + jnp.sum(degp_ref[...][:, 0:n], axis=0)[:, None]
    dis = lax.rsqrt(deg)
    xw = jnp.dot(x_ref[...], w_ref[...], preferred_element_type=jnp.float32)
    y_ref[0:n, :] = xw * dis
    y_ref[n:n_pad, :] = jnp.zeros((n_pad - n, h), jnp.float32)
    dis_ref[...] = dis


def _bn_relu(pre, g_ref, be_ref):
    m = jnp.mean(pre, axis=0, keepdims=True)
    v = jnp.mean((pre - m) ** 2, axis=0, keepdims=True)
    return jax.nn.relu((pre - m) * lax.rsqrt(v + 1e-5) * g_ref[...] + be_ref[...])


def _tc_mid(agg_ref, y_ref, dis_ref, b_ref, g_ref, be_ref, w_ref, out_ref,
            *, n, n_pad, h):
    dis = dis_ref[...]
    pre = dis * (agg_ref[0, 0:n, :] + agg_ref[1, 0:n, :] + y_ref[0:n, :]) + b_ref[...]
    hh = _bn_relu(pre, g_ref, be_ref)
    xw = jnp.dot(hh, w_ref[...], preferred_element_type=jnp.float32)
    out_ref[0:n, :] = xw * dis
    out_ref[n:n_pad, :] = jnp.zeros((n_pad - n, h), jnp.float32)


def _tc_last(agg_ref, y_ref, dis_ref, b_ref, g_ref, be_ref, batch_ref,
             wf1_ref, bf1_ref, wf2_ref, bf2_ref, out_ref, *, n):
    dis = dis_ref[...]
    pre = dis * (agg_ref[0, 0:n, :] + agg_ref[1, 0:n, :] + y_ref[0:n, :]) + b_ref[...]
    hh = _bn_relu(pre, g_ref, be_ref)
    gid = lax.broadcasted_iota(jnp.int32, (NG, n), 0)
    mask = (batch_ref[...] == gid).astype(jnp.float32)
    sums = jnp.dot(mask, hh, preferred_element_type=jnp.float32)
    cnt = jnp.sum(mask, axis=1, keepdims=True)
    pooled = sums / jnp.maximum(cnt, 1.0)
    t = jax.nn.relu(
        jnp.dot(pooled, wf1_ref[...], preferred_element_type=jnp.float32)
        + bf1_ref[...]
    )
    out_ref[...] = (
        jnp.dot(t, wf2_ref[...], preferred_element_type=jnp.float32) + bf2_ref[...]
    )


def kernel(x, edge_index, batch, W1, b1, g1, be1, W2, b2, g2, be2,
           W3, b3, g3, be3, Wf1, bf1, Wf2, bf2):
    n, f_in = x.shape
    h = W1.shape[1]
    e = edge_index.shape[1]
    n_pad = ((n + 1 + NS * 8 - 1) // (NS * 8)) * (NS * 8)

    e_pad = ((e + 8 * NW * CH - 1) // (8 * NW * CH)) * (8 * NW * CH)
    ncw = e_pad // (NW * CH)
    pad = jnp.full((e_pad - e,), n, dtype=jnp.int32)
    srcp = jnp.concatenate([edge_index[0], pad])
    dstp = jnp.concatenate([edge_index[1], pad])
    srcp3 = srcp.reshape(NW, ncw, CH)
    dstp3 = dstp.reshape(NW, ncw, CH)
    zeros_n = jnp.zeros((n_pad, h), jnp.float32)

    agg_h = _make_agg(n_pad, h, ncw, hbm_gather_frac4=2)

    degp = _make_deg(n_pad, ncw)(
        dstp.reshape(NW, ncw * CH), jnp.zeros((n_pad,), jnp.float32)
    )

    y1, dis = pl.pallas_call(
        functools.partial(_tc_first, n=n, n_pad=n_pad, h=h),
        out_shape=[
            jax.ShapeDtypeStruct((n_pad, h), jnp.float32),
            jax.ShapeDtypeStruct((n, 1), jnp.float32),
        ],
    )(x, W1, degp)

    mid = pl.pallas_call(
        functools.partial(_tc_mid, n=n, n_pad=n_pad, h=h),
        out_shape=jax.ShapeDtypeStruct((n_pad, h), jnp.float32),
    )

    a1 = agg_h(y1, srcp3, dstp3, zeros_n)
    y2 = mid(a1, y1, dis, b1.reshape(1, h), g1.reshape(1, h), be1.reshape(1, h), W2)
    a2 = agg_h(y2, srcp3, dstp3, zeros_n)
    y3 = mid(a2, y2, dis, b2.reshape(1, h), g2.reshape(1, h), be2.reshape(1, h), W3)
    a3 = agg_h(y3, srcp3, dstp3, zeros_n)

    out = pl.pallas_call(
        functools.partial(_tc_last, n=n),
        out_shape=jax.ShapeDtypeStruct((NG, 1), jnp.float32),
    )(a3, y3, dis, b3.reshape(1, h), g3.reshape(1, h), be3.reshape(1, h),
      batch.reshape(1, n), Wf1, bf1.reshape(1, -1), Wf2, bf2.reshape(1, 1))
    return out


# speedup vs baseline: 1.3982x; 1.3982x over previous
import functools

import jax
import jax.numpy as jnp
from jax import lax
from jax.experimental import pallas as pl
from jax.experimental.pallas import tpu as pltpu
from jax.experimental.pallas import tpu_sc as plsc

NC = 2
NS = 16
NW = NC * NS
CH = 128
NG = 64


def _make_agg(n_pad, h, ncw, hbm_gather_frac4=0):
    mesh = plsc.VectorSubcoreMesh(
        core_axis_name="c", subcore_axis_name="s", num_cores=NC, num_subcores=NS
    )
    rpt = n_pad // NS

    @functools.partial(
        pl.kernel,
        out_type=jax.ShapeDtypeStruct((NC, n_pad, h), jnp.float32),
        mesh=mesh,
        scratch_types=[
            pltpu.VMEM((ncw, CH), jnp.int32),
            pltpu.VMEM((ncw, CH), jnp.int32),
            pltpu.VMEM((2, CH, h), jnp.float32),
            pltpu.VMEM_SHARED((n_pad, h), jnp.float32),
            pltpu.VMEM_SHARED((n_pad, h), jnp.float32),
        ] + [pltpu.SemaphoreType.DMA] * 4,
        compiler_params=pltpu.CompilerParams(use_tc_tiling_on_sc=False),
    )
    def agg(tab, srcp, dstp, zros, out, src_v, dst_v, rows_v, tab_sh, acc,
            g0, g1, s0, s1):
        c = lax.axis_index("c")
        s = lax.axis_index("s")
        w = s * NC + c
        gsem = (g0, g1)
        ssem = (s0, s1)
        pltpu.sync_copy(srcp.at[w], src_v)
        pltpu.sync_copy(dstp.at[w], dst_v)
        pltpu.sync_copy(tab.at[pl.ds(s * rpt, rpt)], tab_sh.at[pl.ds(s * rpt, rpt)])
        pltpu.sync_copy(zros.at[pl.ds(s * rpt, rpt)], acc.at[pl.ds(s * rpt, rpt)])
        plsc.subcore_barrier()

        def src_tab(t):
            return tab if t % 4 < hbm_gather_frac4 else tab_sh

        def gather(i, b, t):
            pltpu.async_copy(src_tab(t).at[src_v.at[i]], rows_v.at[b], gsem[b])

        def gather_wait(i, b, t):
            pltpu.make_async_copy(
                src_tab(t).at[src_v.at[i]], rows_v.at[b], gsem[b]
            ).wait()

        def scatter(i, b):
            pltpu.async_copy(rows_v.at[b], acc.at[dst_v.at[i]], ssem[b], add=True)

        def scatter_wait(i, b):
            pltpu.make_async_copy(rows_v.at[b], acc.at[dst_v.at[i]], ssem[b]).wait()

        for b in (0, 1):
            gather(b, b, b)

        def step(j, carry):
            i0 = 4 * j
            for half in (0, 2):
                for t in (half, half + 1):
                    b = t % 2
                    gather_wait(i0 + t, b, t)
                    scatter(i0 + t, b)
                for t in (half, half + 1):
                    b = t % 2
                    scatter_wait(i0 + t, b)

                    @pl.when(i0 + t + 2 < ncw)
                    def _():
                        gather(i0 + t + 2, b, t + 2)

            return carry

        lax.fori_loop(0, ncw // 4, step, 0)
        plsc.subcore_barrier()
        pltpu.sync_copy(
            acc.at[pl.ds(s * rpt, rpt)], out.at[c, pl.ds(s * rpt, rpt)]
        )

    return agg


def _make_deg(n_pad, ncw):
    mesh = plsc.VectorSubcoreMesh(
        core_axis_name="c", subcore_axis_name="s", num_cores=NC, num_subcores=NS
    )
    epw = ncw * CH

    @functools.partial(
        pl.kernel,
        out_type=jax.ShapeDtypeStruct((NW, n_pad), jnp.float32),
        mesh=mesh,
        scratch_types=[
            pltpu.VMEM((epw,), jnp.int32),
            pltpu.VMEM((n_pad,), jnp.float32),
        ],
        compiler_params=pltpu.CompilerParams(
            use_tc_tiling_on_sc=False, needs_layout_passes=False
        ),
    )
    def deg(dstp, zros, out, idx_v, hist_v):
        c = lax.axis_index("c")
        s = lax.axis_index("s")
        w = s * NC + c
        pltpu.sync_copy(dstp.at[w], idx_v)
        pltpu.sync_copy(zros, hist_v)
        ones = jnp.ones((16,), jnp.float32)

        def step(k, carry):
            for u in range(8):
                idx16 = idx_v[pl.ds((8 * k + u) * 16, 16)]
                plsc.addupdate_scatter(hist_v, [idx16], ones)
            return carry

        lax.fori_loop(0, epw // 128, step, 0)
        pltpu.sync_copy(hist_v, out.at[w])

    return deg


def _tc_first(x_ref, w_ref, degp_ref, y_ref, dis_ref, *, n, n_pad, h):
    deg = 1.0 + jnp.sum(degp_ref[...][:, 0:n], axis=0)[:, None]
    dis = lax.rsqrt(deg)
    xw = jnp.dot(x_ref[...], w_ref[...], preferred_element_type=jnp.float32)
    y_ref[0:n, :] = xw * dis
    y_ref[n:n_pad, :] = jnp.zeros((n_pad - n, h), jnp.float32)
    dis_ref[...] = dis


def _bn_relu(pre, g_ref, be_ref):
    m = jnp.mean(pre, axis=0, keepdims=True)
    v = jnp.mean((pre - m) ** 2, axis=0, keepdims=True)
    return jax.nn.relu((pre - m) * lax.rsqrt(v + 1e-5) * g_ref[...] + be_ref[...])


def _tc_mid(agg_ref, y_ref, dis_ref, b_ref, g_ref, be_ref, w_ref, out_ref,
            *, n, n_pad, h):
    dis = dis_ref[...]
    pre = dis * (agg_ref[0, 0:n, :] + agg_ref[1, 0:n, :] + y_ref[0:n, :]) + b_ref[...]
    hh = _bn_relu(pre, g_ref, be_ref)
    xw = jnp.dot(hh, w_ref[...], preferred_element_type=jnp.float32)
    out_ref[0:n, :] = xw * dis
    out_ref[n:n_pad, :] = jnp.zeros((n_pad - n, h), jnp.float32)


def _tc_last(agg_ref, y_ref, dis_ref, b_ref, g_ref, be_ref, batch_ref,
             wf1_ref, bf1_ref, wf2_ref, bf2_ref, out_ref, *, n):
    dis = dis_ref[...]
    pre = dis * (agg_ref[0, 0:n, :] + agg_ref[1, 0:n, :] + y_ref[0:n, :]) + b_ref[...]
    hh = _bn_relu(pre, g_ref, be_ref)
    gid = lax.broadcasted_iota(jnp.int32, (NG, n), 0)
    mask = (batch_ref[...] == gid).astype(jnp.float32)
    sums = jnp.dot(mask, hh, preferred_element_type=jnp.float32)
    cnt = jnp.sum(mask, axis=1, keepdims=True)
    pooled = sums / jnp.maximum(cnt, 1.0)
    t = jax.nn.relu(
        jnp.dot(pooled, wf1_ref[...], preferred_element_type=jnp.float32)
        + bf1_ref[...]
    )
    out_ref[...] = (
        jnp.dot(t, wf2_ref[...], preferred_element_type=jnp.float32) + bf2_ref[...]
    )


def kernel(x, edge_index, batch, W1, b1, g1, be1, W2, b2, g2, be2,
           W3, b3, g3, be3, Wf1, bf1, Wf2, bf2):
    n, f_in = x.shape
    h = W1.shape[1]
    e = edge_index.shape[1]
    n_pad = ((n + 1 + NS * 8 - 1) // (NS * 8)) * (NS * 8)

    e_pad = ((e + 8 * NW * CH - 1) // (8 * NW * CH)) * (8 * NW * CH)
    ncw = e_pad // (NW * CH)
    pad = jnp.full((e_pad - e,), n, dtype=jnp.int32)
    srcp = jnp.concatenate([edge_index[0], pad])
    dstp = jnp.concatenate([edge_index[1], pad])
    srcp3 = srcp.reshape(NW, ncw, CH)
    dstp3 = dstp.reshape(NW, ncw, CH)
    zeros_n = jnp.zeros((n_pad, h), jnp.float32)

    agg_h = _make_agg(n_pad, h, ncw)

    degp = _make_deg(n_pad, ncw)(
        dstp.reshape(NW, ncw * CH), jnp.zeros((n_pad,), jnp.float32)
    )

    y1, dis = pl.pallas_call(
        functools.partial(_tc_first, n=n, n_pad=n_pad, h=h),
        out_shape=[
            jax.ShapeDtypeStruct((n_pad, h), jnp.float32),
            jax.ShapeDtypeStruct((n, 1), jnp.float32),
        ],
    )(x, W1, degp)

    mid = pl.pallas_call(
        functools.partial(_tc_mid, n=n, n_pad=n_pad, h=h),
        out_shape=jax.ShapeDtypeStruct((n_pad, h), jnp.float32),
    )

    a1 = agg_h(y1, srcp3, dstp3, zeros_n)
    y2 = mid(a1, y1, dis, b1.reshape(1, h), g1.reshape(1, h), be1.reshape(1, h), W2)
    a2 = agg_h(y2, srcp3, dstp3, zeros_n)
    y3 = mid(a2, y2, dis, b2.reshape(1, h), g2.reshape(1, h), be2.reshape(1, h), W3)
    a3 = agg_h(y3, srcp3, dstp3, zeros_n)

    out = pl.pallas_call(
        functools.partial(_tc_last, n=n),
        out_shape=jax.ShapeDtypeStruct((NG, 1), jnp.float32),
    )(a3, y3, dis, b3.reshape(1, h), g3.reshape(1, h), be3.reshape(1, h),
      batch.reshape(1, n), Wf1, bf1.reshape(1, -1), Wf2, bf2.reshape(1, 1))
    return out
